# Initial kernel scaffold; baseline (speedup 1.0000x reference)
import jax
import jax.numpy as jnp
from jax.experimental import pallas as pl


def kernel(idx, node_emb1, node_emb2, emb1_w, emb2_w, lin1_w, lin1_b, lin2_w, lin2_b):
    raise NotImplementedError("write your pallas kernel here")


# speedup vs baseline: 2.6615x; 2.6615x over previous
import jax
import jax.numpy as jnp
from jax.experimental import pallas as pl
from jax.experimental.pallas import tpu as pltpu

_N = 10000
_DIM = 128
_K = 16
_ALPHA = 3.0


def _row_block(n):
    for rb in (128, 112, 96, 80, 64, 48, 40, 32, 24, 16, 8):
        if n % rb == 0:
            return rb
    return 1


def _nv_kernel(x1_ref, x2_ref, w1t_ref, b1_ref, w2t_ref, b2_ref, nv1_ref, nv2_ref):
    nv1_ref[...] = jnp.tanh(
        _ALPHA * (jnp.dot(x1_ref[...], w1t_ref[...],
                          preferred_element_type=jnp.float32) + b1_ref[...]))
    nv2_ref[...] = jnp.tanh(
        _ALPHA * (jnp.dot(x2_ref[...], w2t_ref[...],
                          preferred_element_type=jnp.float32) + b2_ref[...]))


def _main_kernel(n, rb, nv1b_ref, nv2b_ref, nv1_ref, nv2_ref, noise_ref,
                 adj_ref, v_ref, ew_ref):
    i = pl.program_id(0)
    dn = (((1,), (1,)), ((), ()))
    a = (jax.lax.dot_general(nv1b_ref[...], nv2_ref[...], dn,
                             preferred_element_type=jnp.float32)
         - jax.lax.dot_general(nv2b_ref[...], nv1_ref[...], dn,
                               preferred_element_type=jnp.float32))
    adj = jax.nn.relu(jnp.tanh(_ALPHA * a))
    t = adj + noise_ref[...]
    col = jax.lax.broadcasted_iota(jnp.int32, (rb, n), 1)
    row = i * rb + jax.lax.broadcasted_iota(jnp.int32, (rb, 1), 0)
    keep = jnp.zeros((rb, n), dtype=jnp.bool_)
    neg_inf = jnp.float32(-jnp.inf)
    idx_cols = []
    w_cols = []
    for _ in range(_K):
        m = jnp.max(t, axis=1, keepdims=True)
        idxk = jnp.min(jnp.where(t == m, col, n), axis=1,
                       keepdims=True).astype(jnp.int32)
        onehot = col == idxk
        w = jnp.sum(jnp.where(onehot, adj, 0.0), axis=1)[:, None]
        keep = jnp.logical_or(keep, onehot)
        t = jnp.where(onehot, neg_inf, t)
        idx_cols.append(idxk)
        w_cols.append(w + (idxk == row).astype(jnp.float32))
    eye = (col == row).astype(jnp.float32)
    adj_ref[...] = jnp.where(keep, adj, 0.0) + eye
    v_ref[...] = jnp.concatenate(idx_cols + [row], axis=1)
    ew_ref[...] = jnp.concatenate(
        w_cols + [jnp.ones((rb, 1), dtype=jnp.float32)], axis=1)


def kernel(idx, node_emb1, node_emb2, emb1_w, emb2_w, lin1_w, lin1_b,
           lin2_w, lin2_b):
    n = idx.shape[0]
    dim = emb1_w.shape[1]
    x1 = jnp.take(emb1_w, idx, axis=0)
    x2 = jnp.take(emb2_w, idx, axis=0)
    noise = jax.random.uniform(jax.random.key(1), (n, n),
                               dtype=jnp.float32) * 0.01

    nv1, nv2 = pl.pallas_call(
        _nv_kernel,
        out_shape=(jax.ShapeDtypeStruct((n, dim), jnp.float32),
                   jax.ShapeDtypeStruct((n, dim), jnp.float32)),
    )(x1, x2, lin1_w.T, lin1_b[None, :], lin2_w.T, lin2_b[None, :])

    rb = _row_block(n)
    grid = n // rb
    adj, v, ew = pl.pallas_call(
        lambda *refs: _main_kernel(n, rb, *refs),
        grid=(grid,),
        in_specs=[
            pl.BlockSpec((rb, dim), lambda i: (i, 0)),
            pl.BlockSpec((rb, dim), lambda i: (i, 0)),
            pl.BlockSpec((n, dim), lambda i: (0, 0)),
            pl.BlockSpec((n, dim), lambda i: (0, 0)),
            pl.BlockSpec((rb, n), lambda i: (i, 0)),
        ],
        out_specs=[
            pl.BlockSpec((rb, n), lambda i: (i, 0)),
            pl.BlockSpec((rb, _K + 1), lambda i: (i, 0)),
            pl.BlockSpec((rb, _K + 1), lambda i: (i, 0)),
        ],
        out_shape=(jax.ShapeDtypeStruct((n, n), jnp.float32),
                   jax.ShapeDtypeStruct((n, _K + 1), jnp.int32),
                   jax.ShapeDtypeStruct((n, _K + 1), jnp.float32)),
    )(nv1, nv2, nv1, nv2, noise)

    u = jnp.repeat(jnp.arange(n, dtype=jnp.int32), _K + 1)
    return adj, ew.reshape(-1), u, v.reshape(-1)


# speedup vs baseline: 3.0562x; 1.1483x over previous
import jax
import jax.numpy as jnp
from jax.experimental import pallas as pl
from jax.experimental.pallas import tpu as pltpu

_K = 16
_ALPHA = 3.0


def _nv_kernel(x1_ref, x2_ref, w1t_ref, b1_ref, w2t_ref, b2_ref,
               nv1_ref, nv2_ref):
    nv1_ref[...] = jnp.tanh(
        _ALPHA * (jnp.dot(x1_ref[...], w1t_ref[...],
                          preferred_element_type=jnp.float32) + b1_ref[...]))
    nv2_ref[...] = jnp.tanh(
        _ALPHA * (jnp.dot(x2_ref[...], w2t_ref[...],
                          preferred_element_type=jnp.float32) + b2_ref[...]))


def _noise_at(p):
    u32 = jnp.uint32
    ks0 = u32(0)
    ks1 = u32(1)
    ks2 = u32(0x1BD11BDB)
    ks = (ks0, ks1, ks2)
    x0 = jnp.zeros_like(p, dtype=u32) + ks0
    x1 = p.astype(u32) + ks1

    def rotl(v, d):
        return jnp.left_shift(v, u32(d)) | jnp.right_shift(v, u32(32 - d))

    rot_groups = ((13, 15, 26, 6), (17, 29, 16, 24))
    for g in range(5):
        for r in rot_groups[g % 2]:
            x0 = x0 + x1
            x1 = rotl(x1, r)
            x1 = x0 ^ x1
        x0 = x0 + ks[(g + 1) % 3]
        x1 = x1 + ks[(g + 2) % 3] + u32(g + 1)
    bits = x0 ^ x1
    f = jax.lax.bitcast_convert_type(
        jnp.right_shift(bits, u32(9)) | u32(0x3F800000), jnp.float32)
    return (f - 1.0) * 0.01


def _main_kernel(n, rb, nv1b_ref, nv2b_ref, nv1_ref, nv2_ref,
                 adj_ref, v_ref, ew_ref):
    i = pl.program_id(0)
    dn = (((1,), (1,)), ((), ()))
    col = jax.lax.broadcasted_iota(jnp.int32, (rb, n), 1)
    row = i * rb + jax.lax.broadcasted_iota(jnp.int32, (rb, 1), 0)

    a = (jax.lax.dot_general(nv1b_ref[...], nv2_ref[...], dn,
                             preferred_element_type=jnp.float32)
         - jax.lax.dot_general(nv2b_ref[...], nv1_ref[...], dn,
                               preferred_element_type=jnp.float32))
    adj = jax.nn.relu(jnp.tanh(_ALPHA * a))
    t = adj + _noise_at(row * n + col)

    keep = jnp.zeros((rb, n), dtype=jnp.bool_)
    neg_inf = jnp.float32(-jnp.inf)
    idx_cols = []
    m_cols = []
    for _ in range(_K):
        m = jnp.max(t, axis=1, keepdims=True)
        idxk = jnp.min(jnp.where(t == m, col, n), axis=1,
                       keepdims=True).astype(jnp.int32)
        hit = col == idxk
        keep = jnp.logical_or(keep, hit)
        t = jnp.where(hit, neg_inf, t)
        idx_cols.append(idxk)
        m_cols.append(m)
    idx = jnp.concatenate(idx_cols, axis=1)
    m = jnp.concatenate(m_cols, axis=1)

    eye = (col == row).astype(jnp.float32)
    adj_ref[...] = jnp.where(keep, adj, 0.0) + eye
    w = m - _noise_at(row * n + idx) + (idx == row).astype(jnp.float32)
    v_ref[...] = jnp.concatenate([idx, row], axis=1)
    ew_ref[...] = jnp.concatenate(
        [w, jnp.ones((rb, 1), dtype=jnp.float32)], axis=1)


def _row_block(n):
    for rb in (80, 64, 48, 40, 32, 24, 16, 8):
        if n % rb == 0:
            return rb
    return 1


def kernel(idx, node_emb1, node_emb2, emb1_w, emb2_w, lin1_w, lin1_b,
           lin2_w, lin2_b):
    n = idx.shape[0]
    dim = emb1_w.shape[1]
    x1 = jnp.take(emb1_w, idx, axis=0)
    x2 = jnp.take(emb2_w, idx, axis=0)

    nv1, nv2 = pl.pallas_call(
        _nv_kernel,
        out_shape=(jax.ShapeDtypeStruct((n, dim), jnp.float32),
                   jax.ShapeDtypeStruct((n, dim), jnp.float32)),
    )(x1, x2, lin1_w.T, lin1_b[None, :], lin2_w.T, lin2_b[None, :])

    rb = _row_block(n)
    grid = n // rb
    adj, v, ew = pl.pallas_call(
        lambda *refs: _main_kernel(n, rb, *refs),
        grid=(grid,),
        in_specs=[
            pl.BlockSpec((rb, dim), lambda i: (i, 0)),
            pl.BlockSpec((rb, dim), lambda i: (i, 0)),
            pl.BlockSpec((n, dim), lambda i: (0, 0)),
            pl.BlockSpec((n, dim), lambda i: (0, 0)),
        ],
        out_specs=[
            pl.BlockSpec((rb, n), lambda i: (i, 0)),
            pl.BlockSpec((rb, _K + 1), lambda i: (i, 0)),
            pl.BlockSpec((rb, _K + 1), lambda i: (i, 0)),
        ],
        out_shape=(jax.ShapeDtypeStruct((n, n), jnp.float32),
                   jax.ShapeDtypeStruct((n, _K + 1), jnp.int32),
                   jax.ShapeDtypeStruct((n, _K + 1), jnp.float32)),
    )(nv1, nv2, nv1, nv2)

    u = jnp.repeat(jnp.arange(n, dtype=jnp.int32), _K + 1)
    return adj, ew.reshape(-1), u, v.reshape(-1)


# speedup vs baseline: 3.6627x; 1.1985x over previous
import jax
import jax.numpy as jnp
from jax.experimental import pallas as pl
from jax.experimental.pallas import tpu as pltpu

_K = 16
_ALPHA = 3.0


def _nv_kernel(x1_ref, x2_ref, w1t_ref, b1_ref, w2t_ref, b2_ref,
               nv1_ref, nv2_ref):
    nv1_ref[...] = jnp.tanh(
        _ALPHA * (jnp.dot(x1_ref[...], w1t_ref[...],
                          preferred_element_type=jnp.float32) + b1_ref[...]))
    nv2_ref[...] = jnp.tanh(
        _ALPHA * (jnp.dot(x2_ref[...], w2t_ref[...],
                          preferred_element_type=jnp.float32) + b2_ref[...]))


def _noise_at(p):
    u32 = jnp.uint32
    ks0 = u32(0)
    ks1 = u32(1)
    ks2 = u32(0x1BD11BDB)
    ks = (ks0, ks1, ks2)
    x0 = jnp.zeros_like(p, dtype=u32) + ks0
    x1 = p.astype(u32) + ks1

    def rotl(v, d):
        return jnp.left_shift(v, u32(d)) | jnp.right_shift(v, u32(32 - d))

    rot_groups = ((13, 15, 26, 6), (17, 29, 16, 24))
    for g in range(5):
        for r in rot_groups[g % 2]:
            x0 = x0 + x1
            x1 = rotl(x1, r)
            x1 = x0 ^ x1
        x0 = x0 + ks[(g + 1) % 3]
        x1 = x1 + ks[(g + 2) % 3] + u32(g + 1)
    bits = x0 ^ x1
    f = jax.lax.bitcast_convert_type(
        jnp.right_shift(bits, u32(9)) | u32(0x3F800000), jnp.float32)
    return (f - 1.0) * 0.01


def _main_kernel(n, rb, nv1b_ref, nv2b_ref, nv1_ref, nv2_ref,
                 adj_ref, v_ref, ew_ref):
    i = pl.program_id(0)
    dn = (((1,), (1,)), ((), ()))
    col = jax.lax.broadcasted_iota(jnp.int32, (rb, n), 1)
    row = i * rb + jax.lax.broadcasted_iota(jnp.int32, (rb, 1), 0)

    a = (jax.lax.dot_general(nv1b_ref[...], nv2_ref[...], dn,
                             preferred_element_type=jnp.float32)
         - jax.lax.dot_general(nv2b_ref[...], nv1_ref[...], dn,
                               preferred_element_type=jnp.float32))
    adj = jax.nn.relu(jnp.tanh(_ALPHA * a))
    t = adj + _noise_at(row * n + col)

    neg_inf = jnp.float32(-jnp.inf)
    idx_cols = []
    m_cols = []
    for _ in range(_K):
        m = jnp.max(t, axis=1, keepdims=True)
        idxk = jnp.min(jnp.where(t == m, col, n), axis=1,
                       keepdims=True).astype(jnp.int32)
        t = jnp.where(col == idxk, neg_inf, t)
        idx_cols.append(idxk)
        m_cols.append(m)
    idx = jnp.concatenate(idx_cols, axis=1)
    m = jnp.concatenate(m_cols, axis=1)
    keep = t == neg_inf

    eye = (col == row).astype(jnp.float32)
    adj_ref[...] = jnp.where(keep, adj, 0.0) + eye
    w = m - _noise_at(row * n + idx) + (idx == row).astype(jnp.float32)
    v_ref[...] = jnp.concatenate([idx, row], axis=1)
    ew_ref[...] = jnp.concatenate(
        [w, jnp.ones((rb, 1), dtype=jnp.float32)], axis=1)


def _row_block(n):
    for rb in (80, 64, 48, 40, 32, 24, 16, 8):
        if n % rb == 0:
            return rb
    return 1


def kernel(idx, node_emb1, node_emb2, emb1_w, emb2_w, lin1_w, lin1_b,
           lin2_w, lin2_b):
    n = idx.shape[0]
    dim = emb1_w.shape[1]
    x1 = jnp.take(emb1_w, idx, axis=0)
    x2 = jnp.take(emb2_w, idx, axis=0)

    nv1, nv2 = pl.pallas_call(
        _nv_kernel,
        out_shape=(jax.ShapeDtypeStruct((n, dim), jnp.float32),
                   jax.ShapeDtypeStruct((n, dim), jnp.float32)),
    )(x1, x2, lin1_w.T, lin1_b[None, :], lin2_w.T, lin2_b[None, :])

    rb = _row_block(n)
    grid = n // rb
    adj, v, ew = pl.pallas_call(
        lambda *refs: _main_kernel(n, rb, *refs),
        grid=(grid,),
        in_specs=[
            pl.BlockSpec((rb, dim), lambda i: (i, 0)),
            pl.BlockSpec((rb, dim), lambda i: (i, 0)),
            pl.BlockSpec((n, dim), lambda i: (0, 0)),
            pl.BlockSpec((n, dim), lambda i: (0, 0)),
        ],
        out_specs=[
            pl.BlockSpec((rb, n), lambda i: (i, 0)),
            pl.BlockSpec((rb, _K + 1), lambda i: (i, 0)),
            pl.BlockSpec((rb, _K + 1), lambda i: (i, 0)),
        ],
        out_shape=(jax.ShapeDtypeStruct((n, n), jnp.float32),
                   jax.ShapeDtypeStruct((n, _K + 1), jnp.int32),
                   jax.ShapeDtypeStruct((n, _K + 1), jnp.float32)),
    )(nv1, nv2, nv1, nv2)

    u = jnp.repeat(jnp.arange(n, dtype=jnp.int32), _K + 1)
    return adj, ew.reshape(-1), u, v.reshape(-1)
